# Initial kernel scaffold; baseline (speedup 1.0000x reference)
import jax
import jax.numpy as jnp
from jax.experimental import pallas as pl


def kernel(mem, idx, val):
    raise NotImplementedError("write your pallas kernel here")


# speedup vs baseline: 1.2973x; 1.2973x over previous
import functools

import jax
import jax.numpy as jnp
from jax import lax
from jax.experimental import pallas as pl
from jax.experimental.pallas import tpu as pltpu

M = 262144
D = 128
B = 16384
BLOCK = 1024
GRID = M // BLOCK
LOG2B = 14


def _lower_bound(sidx_ref, target):
    pos = jnp.int32(0)
    for k in (1 << p for p in reversed(range(LOG2B))):
        cand = pos + jnp.int32(k)
        pred = sidx_ref[cand - 1] < target
        pos = jnp.where(pred, cand, pos)
    return jnp.where(sidx_ref[B - 1] < target, jnp.int32(B), pos)


def _block_kernel(sidx_ref, order_ref, mem_ref, val_ref, out_ref):
    i = pl.program_id(0)
    lo = i * BLOCK

    out_ref[...] = mem_ref[...]

    s = _lower_bound(sidx_ref, lo)
    e = _lower_bound(sidx_ref, lo + BLOCK)

    def body(j, carry):
        r = sidx_ref[j] - lo
        o = order_ref[j]
        out_ref[pl.ds(r, 1), :] += val_ref[pl.ds(o, 1), :]
        return carry

    lax.fori_loop(s, e, body, 0)

    x = out_ref[...]
    normsq = jnp.sum(x * x, axis=1, keepdims=True)
    inv = 1.0 / jnp.maximum(jnp.sqrt(normsq), 1e-12)
    out_ref[...] = x * inv


@functools.partial(jax.jit, static_argnames=("interpret",))
def kernel(mem, idx, val, interpret=False):
    idx32 = idx.astype(jnp.int32)
    sidx, order = lax.sort_key_val(idx32, jnp.arange(B, dtype=jnp.int32))

    grid_spec = pltpu.PrefetchScalarGridSpec(
        num_scalar_prefetch=2,
        grid=(GRID,),
        in_specs=[
            pl.BlockSpec((BLOCK, D), lambda i, s_ref, o_ref: (i, 0)),
            pl.BlockSpec((B, D), lambda i, s_ref, o_ref: (0, 0)),
        ],
        out_specs=pl.BlockSpec((BLOCK, D), lambda i, s_ref, o_ref: (i, 0)),
    )
    return pl.pallas_call(
        _block_kernel,
        grid_spec=grid_spec,
        out_shape=jax.ShapeDtypeStruct((M, D), jnp.float32),
        interpret=interpret,
    )(sidx, order, mem, val)
